# baseline (speedup 1.0000x reference)
import functools

import jax
import jax.numpy as jnp
import numpy as np
from jax import lax
from jax.experimental import pallas as pl
from jax.experimental.pallas import tpu as pltpu
from jax.experimental.pallas import tpu_sc as plsc

N_NODES = 10000
N_EDGES = 320000
D_FEAT = 128

NC = 2
NS = 16
NW = NC * NS

NPAD = 10240
EPAD = 327680
CHUNK = 64
IDX_ROWS = EPAD // CHUNK
ROWS_PER_W = IDX_ROWS // NW
NB = 4
IB = 40
ROWS_PER_SUB = NPAD // NS


def _make_sc_agg(with_deg):
    mesh = plsc.VectorSubcoreMesh(core_axis_name="c", subcore_axis_name="s")

    out_type = [jax.ShapeDtypeStruct((NC, NPAD, D_FEAT), jnp.float32)]
    scratch = [
        pltpu.VMEM((IB, CHUNK), jnp.int32),
        pltpu.VMEM((IB, CHUNK), jnp.int32),
        pltpu.VMEM((NB, CHUNK, D_FEAT), jnp.float32),
        pltpu.VMEM_SHARED((NPAD, D_FEAT), jnp.float32),
    ] + [pltpu.SemaphoreType.DMA] * (2 * NB)
    if with_deg:
        out_type.append(jax.ShapeDtypeStruct((NC, NPAD), jnp.float32))
        scratch += [
            pltpu.VMEM((CHUNK,), jnp.float32),
            pltpu.VMEM_SHARED((NPAD,), jnp.float32),
            pltpu.SemaphoreType.DMA,
        ]

    NP = IB // NB

    def body(table, src_r, dst_r, zeros, *rest):
        if with_deg:
            (zerosd, out, out_deg, sidx, didx, rows, acc, *sems) = rest
            gsem, ssem = sems[:NB], sems[NB:2 * NB]
            ones, acc_deg, dsem = sems[2 * NB:]
        else:
            out, sidx, didx, rows, acc, *sems = rest
            gsem, ssem = sems[:NB], sems[NB:2 * NB]

        c = lax.axis_index("c")
        s = lax.axis_index("s")
        wid = c * NS + s

        zsl = pl.ds(s * ROWS_PER_SUB, ROWS_PER_SUB)
        pltpu.sync_copy(zeros, acc.at[zsl])
        if with_deg:
            pltpu.sync_copy(zerosd, acc_deg.at[zsl])
            for i in range(CHUNK // 16):
                ones[pl.ds(16 * i, 16)] = jnp.ones((16,), jnp.float32)
        plsc.subcore_barrier()

        base = wid * ROWS_PER_W

        def gather(i, b):
            pltpu.async_copy(table.at[sidx.at[i]], rows.at[b], gsem[b])

        def scatter(i, b):
            pltpu.async_copy(rows.at[b], acc.at[didx.at[i]], ssem[b], add=True)
            if with_deg:
                pltpu.async_copy(ones, acc_deg.at[didx.at[i]], dsem, add=True)

        def wait_gather(i, b):
            pltpu.make_async_copy(table.at[sidx.at[i]], rows.at[b],
                                  gsem[b]).wait()

        def wait_scatter(i, b):
            pltpu.make_async_copy(rows.at[b], acc.at[didx.at[i]],
                                  ssem[b]).wait()

        def outer(ob, carry):
            r0 = base + ob * IB
            pltpu.sync_copy(src_r.at[pl.ds(r0, IB)], sidx)
            pltpu.sync_copy(dst_r.at[pl.ds(r0, IB)], didx)

            for b in range(NB):
                gather(b, b)

            def quad(gp, c2):
                i0 = gp * NB
                for b in range(NB):
                    wait_gather(i0 + b, b)
                    scatter(i0 + b, b)
                for b in range(NB):
                    wait_scatter(i0 + b, b)
                    gather(i0 + NB + b, b)
                return c2

            lax.fori_loop(0, NP - 1, quad, carry)

            i0 = (NP - 1) * NB
            for b in range(NB):
                wait_gather(i0 + b, b)
                scatter(i0 + b, b)
            for b in range(NB):
                wait_scatter(i0 + b, b)
            if with_deg:
                def drain(i, c3):
                    pltpu.make_async_copy(ones, acc_deg.at[didx.at[0]],
                                          dsem).wait()
                    return c3
                lax.fori_loop(0, IB, drain, carry)
            return carry

        lax.fori_loop(0, ROWS_PER_W // IB, outer, 0)

        plsc.subcore_barrier()
        pltpu.sync_copy(acc.at[zsl], out.at[c, zsl])
        if with_deg:
            pltpu.sync_copy(acc_deg.at[zsl], out_deg.at[c, zsl])

    return functools.partial(
        pl.kernel, mesh=mesh, out_type=out_type, scratch_types=scratch)(body)


_sc_agg_l1 = _make_sc_agg(with_deg=True)
_sc_agg_l2 = _make_sc_agg(with_deg=False)


ROW_BLK = 2048
GRID = NPAD // ROW_BLK


def _tc0_body(deg_ref, inv_ref):
    d = deg_ref[0:1, :] + deg_ref[1:2, :]
    inv_ref[...] = 1.0 / jnp.maximum(d, 1.0)


def _tc1_body(x_ref, sc_ref, inv_ref, ws_ref, wn_ref, b_ref, h_ref):
    agg = sc_ref[0] + sc_ref[1]
    mean = agg * inv_ref[...]
    h = (jnp.dot(x_ref[...], ws_ref[...], preferred_element_type=jnp.float32)
         + jnp.dot(mean, wn_ref[...], preferred_element_type=jnp.float32)
         + b_ref[...])
    h_ref[...] = jax.nn.sigmoid(h)


def _tc2_body(h1_ref, sc_ref, inv_ref, w2s_ref, w2n_ref, b2_ref,
              wm1_ref, bm1_ref, wm2_ref, bm2_ref, out_ref):
    agg = sc_ref[0] + sc_ref[1]
    h1 = h1_ref[...]
    h2 = jax.nn.sigmoid(
        jnp.dot(h1, w2s_ref[...], preferred_element_type=jnp.float32)
        + jnp.dot(agg * inv_ref[...], w2n_ref[...],
                  preferred_element_type=jnp.float32)
        + b2_ref[...])
    t = jnp.maximum(
        jnp.dot(h2, wm1_ref[...], preferred_element_type=jnp.float32)
        + bm1_ref[...], 0.0)
    out_ref[...] = (jnp.dot(t, wm2_ref[...], preferred_element_type=jnp.float32)
                    + bm2_ref[...])


def _full(shape):
    return pl.BlockSpec(shape, lambda i: tuple(0 for _ in shape))


def kernel(features, edge_index, W1_self, W1_neigh, b1, W2_self, W2_neigh, b2,
           Wm1, bm1, Wm2, bm2):
    src = jnp.asarray(edge_index[0], jnp.int32)
    dst = jnp.asarray(edge_index[1], jnp.int32)
    pad_i = np.arange(EPAD - N_EDGES, dtype=np.int32)
    src_r = jnp.concatenate(
        [src, jnp.asarray(pad_i % N_NODES)]).reshape(IDX_ROWS, CHUNK)
    dst_r = jnp.concatenate(
        [dst, jnp.asarray(N_NODES + pad_i % (NPAD - N_NODES))]).reshape(
            IDX_ROWS, CHUNK)

    z2 = jnp.zeros((ROWS_PER_SUB, D_FEAT), jnp.float32)
    zd = jnp.zeros((ROWS_PER_SUB,), jnp.float32)

    sc1, deg2 = _sc_agg_l1(features, src_r, dst_r, z2, zd)

    inv_row = pl.pallas_call(
        _tc0_body,
        grid=(1,),
        in_specs=[_full((NC, NPAD))],
        out_specs=_full((1, NPAD)),
        out_shape=jax.ShapeDtypeStruct((1, NPAD), jnp.float32),
    )(deg2)
    inv_col = inv_row.reshape(NPAD, 1)

    h1 = pl.pallas_call(
        _tc1_body,
        grid=(GRID,),
        in_specs=[
            pl.BlockSpec((ROW_BLK, D_FEAT), lambda i: (i, 0)),
            pl.BlockSpec((NC, ROW_BLK, D_FEAT), lambda i: (0, i, 0)),
            pl.BlockSpec((ROW_BLK, 1), lambda i: (i, 0)),
            _full((D_FEAT, D_FEAT)),
            _full((D_FEAT, D_FEAT)),
            _full((1, D_FEAT)),
        ],
        out_specs=pl.BlockSpec((ROW_BLK, D_FEAT), lambda i: (i, 0)),
        out_shape=jax.ShapeDtypeStruct((N_NODES, D_FEAT), jnp.float32),
    )(features, sc1, inv_col, W1_self, W1_neigh, b1.reshape(1, D_FEAT))

    sc2, = _sc_agg_l2(h1, src_r, dst_r, z2)

    out = pl.pallas_call(
        _tc2_body,
        grid=(GRID,),
        in_specs=[
            pl.BlockSpec((ROW_BLK, D_FEAT), lambda i: (i, 0)),
            pl.BlockSpec((NC, ROW_BLK, D_FEAT), lambda i: (0, i, 0)),
            pl.BlockSpec((ROW_BLK, 1), lambda i: (i, 0)),
            _full((D_FEAT, D_FEAT)),
            _full((D_FEAT, D_FEAT)),
            _full((1, D_FEAT)),
            _full((D_FEAT, D_FEAT)),
            _full((1, D_FEAT)),
            _full((D_FEAT, 64)),
            _full((1, 64)),
        ],
        out_specs=pl.BlockSpec((ROW_BLK, 64), lambda i: (i, 0)),
        out_shape=jax.ShapeDtypeStruct((N_NODES, 64), jnp.float32),
    )(h1, sc2, inv_col, W2_self, W2_neigh, b2.reshape(1, D_FEAT),
      Wm1, bm1.reshape(1, D_FEAT), Wm2, bm2.reshape(1, 64))

    return out


# speedup vs baseline: 1.0388x; 1.0388x over previous
import functools

import jax
import jax.numpy as jnp
import numpy as np
from jax import lax
from jax.experimental import pallas as pl
from jax.experimental.pallas import tpu as pltpu
from jax.experimental.pallas import tpu_sc as plsc

N_NODES = 10000
N_EDGES = 320000
D_FEAT = 128

NC = 2
NS = 16
NW = NC * NS

NPAD = 10240
EPAD = 327680
CHUNK = 64
IDX_ROWS = EPAD // CHUNK
ROWS_PER_W = IDX_ROWS // NW
NB = 4
IB = 16
ROWS_PER_SUB = NPAD // NS


def _make_sc_agg(with_deg):
    mesh = plsc.VectorSubcoreMesh(core_axis_name="c", subcore_axis_name="s")

    NBLK = ROWS_PER_W // IB
    NP = IB // NB

    out_type = [jax.ShapeDtypeStruct((NC, NPAD, D_FEAT), jnp.float32)]
    scratch = [
        pltpu.VMEM((2, IB, CHUNK), jnp.int32),
        pltpu.VMEM((2, IB, CHUNK), jnp.int32),
        pltpu.VMEM((NB, CHUNK, D_FEAT), jnp.float32),
        pltpu.VMEM_SHARED((NPAD, D_FEAT), jnp.float32),
    ] + [pltpu.SemaphoreType.DMA] * (2 * NB + 2)
    if with_deg:
        out_type.append(jax.ShapeDtypeStruct((NC, NPAD), jnp.float32))
        scratch += [
            pltpu.VMEM((CHUNK,), jnp.float32),
            pltpu.VMEM_SHARED((NPAD,), jnp.float32),
            pltpu.SemaphoreType.DMA,
        ]

    def body(table, src_r, dst_r, zeros, *rest):
        if with_deg:
            (zerosd, out, out_deg, sidx, didx, rows, acc, *sems) = rest
            gsem, ssem = sems[:NB], sems[NB:2 * NB]
            isem = sems[2 * NB:2 * NB + 2]
            ones, acc_deg, dsem = sems[2 * NB + 2:]
        else:
            out, sidx, didx, rows, acc, *sems = rest
            gsem, ssem = sems[:NB], sems[NB:2 * NB]
            isem = sems[2 * NB:2 * NB + 2]

        c = lax.axis_index("c")
        s = lax.axis_index("s")
        wid = c * NS + s

        zsl = pl.ds(s * ROWS_PER_SUB, ROWS_PER_SUB)
        pltpu.sync_copy(zeros, acc.at[zsl])
        if with_deg:
            pltpu.sync_copy(zerosd, acc_deg.at[zsl])
            for i in range(CHUNK // 16):
                ones[pl.ds(16 * i, 16)] = jnp.ones((16,), jnp.float32)
        plsc.subcore_barrier()

        base = wid * ROWS_PER_W

        def stage(ob, wait):
            p = ob % 2
            r0 = base + ob * IB
            g = pltpu.async_copy(src_r.at[pl.ds(r0, IB)], sidx.at[p], isem[p])
            h = pltpu.async_copy(dst_r.at[pl.ds(r0, IB)], didx.at[p], isem[p])
            if wait:
                g.wait()
                h.wait()

        def wait_stage(ob):
            p = ob % 2
            r0 = base + ob * IB
            pltpu.make_async_copy(src_r.at[pl.ds(r0, IB)], sidx.at[p],
                                  isem[p]).wait()
            pltpu.make_async_copy(dst_r.at[pl.ds(r0, IB)], didx.at[p],
                                  isem[p]).wait()

        def gather(p, i, b):
            pltpu.async_copy(table.at[sidx.at[p, i]], rows.at[b], gsem[b])

        def scatter(p, i, b):
            pltpu.async_copy(rows.at[b], acc.at[didx.at[p, i]], ssem[b],
                             add=True)
            if with_deg:
                pltpu.async_copy(ones, acc_deg.at[didx.at[p, i]], dsem,
                                 add=True)

        def wait_gather(p, i, b):
            pltpu.make_async_copy(table.at[sidx.at[p, i]], rows.at[b],
                                  gsem[b]).wait()

        def wait_scatter(p, i, b):
            pltpu.make_async_copy(rows.at[b], acc.at[didx.at[p, i]],
                                  ssem[b]).wait()

        def drain_deg(n):
            def dwait(i, c3):
                pltpu.make_async_copy(ones, acc_deg.at[didx.at[0, 0]],
                                      dsem).wait()
                return c3
            lax.fori_loop(0, n, dwait, 0)

        stage(0, wait=True)
        for b in range(NB):
            gather(0, b, b)

        for ob in range(NBLK):
            p = ob % 2
            if ob + 1 < NBLK:
                if with_deg and ob >= 1:
                    drain_deg(IB)
                stage(ob + 1, wait=False)

            def quad(gp, c2):
                i0 = gp * NB
                for b in range(NB):
                    wait_gather(p, i0 + b, b)
                    scatter(p, i0 + b, b)
                for b in range(NB):
                    wait_scatter(p, i0 + b, b)
                    gather(p, i0 + NB + b, b)
                return c2

            lax.fori_loop(0, NP - 1, quad, 0)

            i0 = (NP - 1) * NB
            for b in range(NB):
                wait_gather(p, i0 + b, b)
                scatter(p, i0 + b, b)
            if ob + 1 < NBLK:
                wait_stage(ob + 1)
                for b in range(NB):
                    wait_scatter(p, i0 + b, b)
                    gather((ob + 1) % 2, b, b)
            else:
                for b in range(NB):
                    wait_scatter(p, i0 + b, b)
                if with_deg:
                    drain_deg(2 * IB if NBLK > 1 else IB)

        plsc.subcore_barrier()
        pltpu.sync_copy(acc.at[zsl], out.at[c, zsl])
        if with_deg:
            pltpu.sync_copy(acc_deg.at[zsl], out_deg.at[c, zsl])

    return functools.partial(
        pl.kernel, mesh=mesh, out_type=out_type, scratch_types=scratch)(body)


_sc_agg_l1 = _make_sc_agg(with_deg=True)
_sc_agg_l2 = _make_sc_agg(with_deg=False)


ROW_BLK = 2048
GRID = NPAD // ROW_BLK


def _tc0_body(deg_ref, inv_ref):
    d = deg_ref[0:1, :] + deg_ref[1:2, :]
    inv_ref[...] = 1.0 / jnp.maximum(d, 1.0)


def _tc1_body(x_ref, sc_ref, inv_ref, ws_ref, wn_ref, b_ref, h_ref):
    agg = sc_ref[0] + sc_ref[1]
    mean = agg * inv_ref[...]
    h = (jnp.dot(x_ref[...], ws_ref[...], preferred_element_type=jnp.float32)
         + jnp.dot(mean, wn_ref[...], preferred_element_type=jnp.float32)
         + b_ref[...])
    h_ref[...] = jax.nn.sigmoid(h)


def _tc2_body(h1_ref, sc_ref, inv_ref, w2s_ref, w2n_ref, b2_ref,
              wm1_ref, bm1_ref, wm2_ref, bm2_ref, out_ref):
    agg = sc_ref[0] + sc_ref[1]
    h1 = h1_ref[...]
    h2 = jax.nn.sigmoid(
        jnp.dot(h1, w2s_ref[...], preferred_element_type=jnp.float32)
        + jnp.dot(agg * inv_ref[...], w2n_ref[...],
                  preferred_element_type=jnp.float32)
        + b2_ref[...])
    t = jnp.maximum(
        jnp.dot(h2, wm1_ref[...], preferred_element_type=jnp.float32)
        + bm1_ref[...], 0.0)
    out_ref[...] = (jnp.dot(t, wm2_ref[...], preferred_element_type=jnp.float32)
                    + bm2_ref[...])


def _full(shape):
    return pl.BlockSpec(shape, lambda i: tuple(0 for _ in shape))


def kernel(features, edge_index, W1_self, W1_neigh, b1, W2_self, W2_neigh, b2,
           Wm1, bm1, Wm2, bm2):
    src = jnp.asarray(edge_index[0], jnp.int32)
    dst = jnp.asarray(edge_index[1], jnp.int32)
    pad_i = np.arange(EPAD - N_EDGES, dtype=np.int32)
    src_r = jnp.concatenate(
        [src, jnp.asarray(pad_i % N_NODES)]).reshape(IDX_ROWS, CHUNK)
    dst_r = jnp.concatenate(
        [dst, jnp.asarray(N_NODES + pad_i % (NPAD - N_NODES))]).reshape(
            IDX_ROWS, CHUNK)

    z2 = jnp.zeros((ROWS_PER_SUB, D_FEAT), jnp.float32)
    zd = jnp.zeros((ROWS_PER_SUB,), jnp.float32)

    sc1, deg2 = _sc_agg_l1(features, src_r, dst_r, z2, zd)

    inv_row = pl.pallas_call(
        _tc0_body,
        grid=(1,),
        in_specs=[_full((NC, NPAD))],
        out_specs=_full((1, NPAD)),
        out_shape=jax.ShapeDtypeStruct((1, NPAD), jnp.float32),
    )(deg2)
    inv_col = inv_row.reshape(NPAD, 1)

    h1 = pl.pallas_call(
        _tc1_body,
        grid=(GRID,),
        in_specs=[
            pl.BlockSpec((ROW_BLK, D_FEAT), lambda i: (i, 0)),
            pl.BlockSpec((NC, ROW_BLK, D_FEAT), lambda i: (0, i, 0)),
            pl.BlockSpec((ROW_BLK, 1), lambda i: (i, 0)),
            _full((D_FEAT, D_FEAT)),
            _full((D_FEAT, D_FEAT)),
            _full((1, D_FEAT)),
        ],
        out_specs=pl.BlockSpec((ROW_BLK, D_FEAT), lambda i: (i, 0)),
        out_shape=jax.ShapeDtypeStruct((N_NODES, D_FEAT), jnp.float32),
    )(features, sc1, inv_col, W1_self, W1_neigh, b1.reshape(1, D_FEAT))

    sc2, = _sc_agg_l2(h1, src_r, dst_r, z2)

    out = pl.pallas_call(
        _tc2_body,
        grid=(GRID,),
        in_specs=[
            pl.BlockSpec((ROW_BLK, D_FEAT), lambda i: (i, 0)),
            pl.BlockSpec((NC, ROW_BLK, D_FEAT), lambda i: (0, i, 0)),
            pl.BlockSpec((ROW_BLK, 1), lambda i: (i, 0)),
            _full((D_FEAT, D_FEAT)),
            _full((D_FEAT, D_FEAT)),
            _full((1, D_FEAT)),
            _full((D_FEAT, D_FEAT)),
            _full((1, D_FEAT)),
            _full((D_FEAT, 64)),
            _full((1, 64)),
        ],
        out_specs=pl.BlockSpec((ROW_BLK, 64), lambda i: (i, 0)),
        out_shape=jax.ShapeDtypeStruct((N_NODES, 64), jnp.float32),
    )(h1, sc2, inv_col, W2_self, W2_neigh, b2.reshape(1, D_FEAT),
      Wm1, bm1.reshape(1, D_FEAT), Wm2, bm2.reshape(1, 64))

    return out
